# baseline (speedup 1.0000x reference)
import jax
import jax.numpy as jnp
from jax import lax
from jax.experimental import pallas as pl

N_ROWS = 128
N_COLS = 32768
K = 64
ROW_BLOCK = 8


def _topk_block(x_ref, vals_ref, idxs_ref):
    x = x_ref[...]
    col = lax.broadcasted_iota(jnp.int32, (ROW_BLOCK, N_COLS), 1)
    col_k = lax.broadcasted_iota(jnp.int32, (ROW_BLOCK, K), 1)
    neg_inf = jnp.float32(-jnp.inf)
    vals0 = jnp.zeros((ROW_BLOCK, K), jnp.float32)
    idxs0 = jnp.zeros((ROW_BLOCK, K), jnp.int32)

    def step(j, carry):
        x, vals, idxs = carry
        vmax = jnp.max(x, axis=1, keepdims=True)
        eq = x >= vmax
        idx = jnp.min(jnp.where(eq, col, jnp.int32(N_COLS)), axis=1, keepdims=True)
        vals = jnp.where(col_k == j, vmax, vals)
        idxs = jnp.where(col_k == j, idx, idxs)
        x = jnp.where(col == idx, neg_inf, x)
        return (x, vals, idxs)

    _, vals, idxs = lax.fori_loop(0, K, step, (x, vals0, idxs0))
    vals_ref[...] = vals
    idxs_ref[...] = idxs


def kernel(inputs):
    grid = (N_ROWS // ROW_BLOCK,)
    vals, idxs = pl.pallas_call(
        _topk_block,
        grid=grid,
        in_specs=[pl.BlockSpec((ROW_BLOCK, N_COLS), lambda i: (i, 0))],
        out_specs=[
            pl.BlockSpec((ROW_BLOCK, K), lambda i: (i, 0)),
            pl.BlockSpec((ROW_BLOCK, K), lambda i: (i, 0)),
        ],
        out_shape=[
            jax.ShapeDtypeStruct((N_ROWS, K), jnp.float32),
            jax.ShapeDtypeStruct((N_ROWS, K), jnp.int32),
        ],
    )(inputs)
    return (vals, idxs)


# speedup vs baseline: 3.1134x; 3.1134x over previous
import dataclasses
import functools

import jax
import jax.numpy as jnp
import numpy as np
from jax import lax
from jax.experimental import pallas as pl
from jax.experimental.pallas import tpu as pltpu
from jax.experimental.pallas import tpu_sc as plsc

N_ROWS = 128
N_COLS = 32768
K = 64
NC = 2
NS = 16
NW = NC * NS
RPW = N_ROWS // NW
NV = N_COLS // 16
SIGN = np.int32(-2147483648)
MAGN = np.int32(0x7FFFFFFF)


def _key(x):
    u = plsc.bitcast(x, jnp.int32)
    s = lax.shift_right_arithmetic(u, 31)
    return lax.bitwise_xor(u, lax.bitwise_and(s, MAGN))


def _topk_body(x_hbm, vals_hbm, idx_hbm,
               raw_v, cand_v, hist_v, outv_v, outi_v,
               sk_v, sp_v, sortv_v, sorti_v, ks_v, ps_v):
    cid = lax.axis_index("c")
    sid = lax.axis_index("s")
    wid = sid * NC + cid
    lanes = lax.iota(jnp.int32, 16)
    zeros16 = lanes - lanes
    ones = zeros16 + 1

    def zero_hist():
        @pl.loop(0, 256)
        def _z(i):
            hist_v[pl.ds(i * 16, 16)] = zeros16

    def sweep(r):
        def cond(c):
            _, acc, _ = c
            return acc < r

        def body(c):
            b, acc, _ = c
            b = b - 1
            t = jnp.sum(hist_v[pl.ds(b * 16, 16)])
            return b, acc + t, t

        b, acc, t = lax.while_loop(
            cond, body, (np.int32(256), np.int32(0), np.int32(0)))
        return b, acc - t, t

    @pl.loop(0, RPW)
    def _row(t):
        row = wid * RPW + t
        pltpu.sync_copy(x_hbm.at[row], raw_v)

        zero_hist()

        @pl.loop(0, NV)
        def _s1(i):
            x = raw_v[pl.ds(i * 16, 16)]
            kb = lax.bitwise_xor(_key(x), SIGN)
            d = lax.shift_right_logical(kb, 24)
            plsc.addupdate_scatter(hist_v, [d * 16 + lanes], ones)

        b1, g1, t1 = sweep(np.int32(K))

        def s2(i, off):
            x = raw_v[pl.ds(i * 16, 16)]
            kb = lax.bitwise_xor(_key(x), SIGN)
            d = lax.shift_right_logical(kb, 24)
            m = d >= b1
            pos = off + plsc.cumsum(m.astype(jnp.int32)) - 1
            plsc.store_scatter(cand_v, [pos], i * 16 + lanes, mask=m)
            return off + plsc.all_reduce_population_count(m)

        off = lax.fori_loop(0, NV, s2, zeros16)
        n = jnp.max(off)
        nvc = lax.shift_right_logical(n + 15, 4)

        def refine(B, r, cnt, shift):
            zero_hist()

            def rb(i, _):
                base = i * 16
                valid = (base + lanes) < n
                cidx = cand_v[pl.ds(base, 16)]
                x = plsc.load_gather(raw_v, [cidx], mask=valid)
                kb = lax.bitwise_xor(_key(x), SIGN)
                pref = lax.shift_right_logical(kb, shift + 8)
                tie = jnp.logical_and(pref == B, valid)
                d = lax.bitwise_and(
                    lax.shift_right_logical(kb, shift), np.int32(0xFF))
                plsc.addupdate_scatter(hist_v, [d * 16 + lanes], ones, mask=tie)
                return 0

            lax.fori_loop(0, nvc, rb, 0)
            b, g, tb = sweep(r)
            return B * 256 + b, r - g, tb

        B, r, cnt = b1, np.int32(K) - g1, t1
        for shift in (16, 8, 0):
            def _skip(B, r, cnt):
                return B * 256, r, cnt

            def _do(B, r, cnt, _s=shift):
                return refine(B, r, cnt, _s)

            B, r, cnt = lax.cond(r == cnt, _skip, _do, B, r, cnt)

        take_all = r == cnt
        ts = lax.bitwise_xor(B, SIGN)
        tcut = jnp.where(take_all, ts - 1, ts)
        r_tie = jnp.where(take_all, 0, r)

        def fin(i, carry):
            off_o, eq_seen = carry
            base = i * 16
            valid = (base + lanes) < n
            cidx = cand_v[pl.ds(base, 16)]
            x = plsc.load_gather(raw_v, [cidx], mask=valid)
            ms = _key(x)
            gt = jnp.logical_and(ms > tcut, valid)
            eq = jnp.logical_and(jnp.logical_and(ms == ts, valid),
                                 jnp.logical_not(gt))
            eq_rank = eq_seen + plsc.cumsum(eq.astype(jnp.int32))
            sel = jnp.logical_or(gt, jnp.logical_and(eq, eq_rank <= r_tie))
            pos = off_o + plsc.cumsum(sel.astype(jnp.int32)) - 1
            plsc.store_scatter(outv_v, [pos], x, mask=sel)
            plsc.store_scatter(outi_v, [pos], cidx, mask=sel)
            return (off_o + plsc.all_reduce_population_count(sel),
                    eq_seen + plsc.all_reduce_population_count(eq))

        lax.fori_loop(0, nvc, fin, (zeros16, zeros16))

        def ce(ka, pa, kb, pb):
            c = ka >= kb
            return (jnp.where(c, ka, kb), jnp.where(c, pa, pb),
                    jnp.where(c, kb, ka), jnp.where(c, pb, pa))

        def rev(k, p):
            return lax.rev(k, (0,)), lax.rev(p, (0,))

        def perm_gather(k, p, pidx):
            sk_v[...] = k
            sp_v[...] = p
            return (plsc.load_gather(sk_v, [pidx]),
                    plsc.load_gather(sp_v, [pidx]))

        def clean(k, p):
            for j in (8, 4, 2, 1):
                pidx = lax.bitwise_xor(lanes, np.int32(j))
                kp, pp = perm_gather(k, p, pidx)
                is_lo = lax.bitwise_and(lanes, np.int32(j)) == 0
                keep = jnp.where(is_lo, k >= kp, kp >= k)
                k = jnp.where(keep, k, kp)
                p = jnp.where(keep, p, pp)
            return k, p

        def merge32(ka, pa, kb, pb):
            kb, pb = rev(kb, pb)
            ka, pa, kb, pb = ce(ka, pa, kb, pb)
            ka, pa = clean(ka, pa)
            kb, pb = clean(kb, pb)
            return ka, pa, kb, pb

        ks, ps = [], []
        for v in range(4):
            xv = outv_v[pl.ds(v * 16, 16)]
            kv, pv = plsc.sort_key_val(_key(xv), v * 16 + lanes,
                                       descending=True)
            ks.append(kv)
            ps.append(pv)

        ks[0], ps[0], ks[1], ps[1] = merge32(ks[0], ps[0], ks[1], ps[1])
        ks[2], ps[2], ks[3], ps[3] = merge32(ks[2], ps[2], ks[3], ps[3])

        rk3, rp3 = rev(ks[3], ps[3])
        rk2, rp2 = rev(ks[2], ps[2])
        k0, p0, rk3, rp3 = ce(ks[0], ps[0], rk3, rp3)
        k1, p1, rk2, rp2 = ce(ks[1], ps[1], rk2, rp2)
        k0, p0, k1, p1 = ce(k0, p0, k1, p1)
        rk3, rp3, rk2, rp2 = ce(rk3, rp3, rk2, rp2)
        k0, p0 = clean(k0, p0)
        k1, p1 = clean(k1, p1)
        k2, p2 = clean(rk3, rp3)
        k3, p3 = clean(rk2, rp2)

        kall = (k0, k1, k2, k3)
        for v in range(4):
            ks_v[pl.ds(v * 16, 16)] = kall[v]
        ps = [p0, p1, p2, p3]
        for q in (0, 1, 0, 1):
            for v in range(4):
                ps_v[pl.ds(v * 16, 16)] = ps[v]
            new_ps = []
            for v in range(4):
                e = v * 16 + lanes
                if q == 0:
                    partner = lax.bitwise_xor(e, 1)
                else:
                    partner = jnp.clip(lax.bitwise_xor(e + 1, 1) - 1, 0, 63)
                kp = plsc.load_gather(ks_v, [partner])
                pp = plsc.load_gather(ps_v, [partner])
                k, p = kall[v], ps[v]
                take = jnp.logical_and(
                    k == kp,
                    jnp.where(partner > e, pp < p, pp > p))
                new_ps.append(jnp.where(take, pp, p))
            ps = new_ps

        for v, pv in enumerate(ps):
            sortv_v[pl.ds(v * 16, 16)] = plsc.load_gather(outv_v, [pv])
            sorti_v[pl.ds(v * 16, 16)] = plsc.load_gather(outi_v, [pv])

        pltpu.sync_copy(sortv_v, vals_hbm.at[row])
        pltpu.sync_copy(sorti_v, idx_hbm.at[row])


@jax.jit
def _sc_topk(inputs):
    mesh = plsc.VectorSubcoreMesh(core_axis_name="c", subcore_axis_name="s")
    cp = pltpu.CompilerParams()
    if "needs_layout_passes" in pltpu.CompilerParams.__dataclass_fields__:
        cp = dataclasses.replace(cp, needs_layout_passes=False)
    f = pl.kernel(
        _topk_body,
        compiler_params=cp,
        out_type=[
            jax.ShapeDtypeStruct((N_ROWS, K), jnp.float32),
            jax.ShapeDtypeStruct((N_ROWS, K), jnp.int32),
        ],
        mesh=mesh,
        scratch_types=[
            pltpu.VMEM((N_COLS,), jnp.float32),
            pltpu.VMEM((N_COLS,), jnp.int32),
            pltpu.VMEM((256 * 16,), jnp.int32),
            pltpu.VMEM((K,), jnp.float32),
            pltpu.VMEM((K,), jnp.int32),
            pltpu.VMEM((16,), jnp.int32),
            pltpu.VMEM((16,), jnp.int32),
            pltpu.VMEM((K,), jnp.float32),
            pltpu.VMEM((K,), jnp.int32),
            pltpu.VMEM((K,), jnp.int32),
            pltpu.VMEM((K,), jnp.int32),
        ],
    )
    return f(inputs)


def kernel(inputs):
    vals, idxs = _sc_topk(inputs)
    return (vals, idxs)


# speedup vs baseline: 3.1712x; 1.0186x over previous
import dataclasses
import functools

import jax
import jax.numpy as jnp
import numpy as np
from jax import lax
from jax.experimental import pallas as pl
from jax.experimental.pallas import tpu as pltpu
from jax.experimental.pallas import tpu_sc as plsc

N_ROWS = 128
N_COLS = 32768
K = 64
NC = 2
NS = 16
NW = NC * NS
RPW = N_ROWS // NW
NV = N_COLS // 16
SIGN = np.int32(-2147483648)
MAGN = np.int32(0x7FFFFFFF)


def _key(x):
    u = plsc.bitcast(x, jnp.int32)
    s = lax.shift_right_arithmetic(u, 31)
    return lax.bitwise_xor(u, lax.bitwise_and(s, MAGN))


def _topk_body(x_hbm, vals_hbm, idx_hbm,
               raw_v, cand_v, hist_v, outv_v, outi_v,
               sk_v, sp_v, sortv_v, sorti_v, ks_v, ps_v):
    cid = lax.axis_index("c")
    sid = lax.axis_index("s")
    wid = sid * NC + cid
    lanes = lax.iota(jnp.int32, 16)
    zeros16 = lanes - lanes
    ones = zeros16 + 1

    def zero_hist():
        @pl.loop(0, 256)
        def _z(i):
            hist_v[pl.ds(i * 16, 16)] = zeros16

    def sweep(r):
        def cond(c):
            _, acc, _ = c
            return acc < r

        def body(c):
            b, acc, _ = c
            b = b - 1
            t = jnp.sum(hist_v[pl.ds(b * 16, 16)])
            return b, acc + t, t

        b, acc, t = lax.while_loop(
            cond, body, (np.int32(256), np.int32(0), np.int32(0)))
        return b, acc - t, t

    @pl.loop(0, RPW)
    def _row(t):
        row = wid * RPW + t
        pltpu.sync_copy(x_hbm.at[row], raw_v)

        zero_hist()

        @pl.loop(0, NV, step=4)
        def _s1(i):
            for u in range(4):
                x = raw_v[pl.ds((i + u) * 16, 16)]
                kb = lax.bitwise_xor(_key(x), SIGN)
                d = lax.shift_right_logical(kb, 24)
                plsc.addupdate_scatter(hist_v, [d * 16 + lanes], ones)

        b1, g1, t1 = sweep(np.int32(K))

        def s2(i, off):
            for u in range(4):
                x = raw_v[pl.ds((i * 4 + u) * 16, 16)]
                kb = lax.bitwise_xor(_key(x), SIGN)
                d = lax.shift_right_logical(kb, 24)
                m = d >= b1
                pos = off + plsc.cumsum(m.astype(jnp.int32)) - 1
                plsc.store_scatter(cand_v, [pos], (i * 4 + u) * 16 + lanes,
                                   mask=m)
                off = off + plsc.all_reduce_population_count(m)
            return off

        off = lax.fori_loop(0, NV // 4, s2, zeros16)
        n = jnp.max(off)
        nvc = lax.shift_right_logical(n + 15, 4)

        def refine(B, r, cnt, shift):
            zero_hist()

            def rb(i, _):
                base = i * 16
                valid = (base + lanes) < n
                cidx = cand_v[pl.ds(base, 16)]
                x = plsc.load_gather(raw_v, [cidx], mask=valid)
                kb = lax.bitwise_xor(_key(x), SIGN)
                pref = lax.shift_right_logical(kb, shift + 8)
                tie = jnp.logical_and(pref == B, valid)
                d = lax.bitwise_and(
                    lax.shift_right_logical(kb, shift), np.int32(0xFF))
                plsc.addupdate_scatter(hist_v, [d * 16 + lanes], ones, mask=tie)
                return 0

            lax.fori_loop(0, nvc, rb, 0)
            b, g, tb = sweep(r)
            return B * 256 + b, r - g, tb

        B, r, cnt = b1, np.int32(K) - g1, t1
        for shift in (16, 8, 0):
            def _skip(B, r, cnt):
                return B * 256, r, cnt

            def _do(B, r, cnt, _s=shift):
                return refine(B, r, cnt, _s)

            B, r, cnt = lax.cond(r == cnt, _skip, _do, B, r, cnt)

        take_all = r == cnt
        ts = lax.bitwise_xor(B, SIGN)
        tcut = jnp.where(take_all, ts - 1, ts)
        r_tie = jnp.where(take_all, 0, r)

        def fin(i, carry):
            off_o, eq_seen = carry
            base = i * 16
            valid = (base + lanes) < n
            cidx = cand_v[pl.ds(base, 16)]
            x = plsc.load_gather(raw_v, [cidx], mask=valid)
            ms = _key(x)
            gt = jnp.logical_and(ms > tcut, valid)
            eq = jnp.logical_and(jnp.logical_and(ms == ts, valid),
                                 jnp.logical_not(gt))
            eq_rank = eq_seen + plsc.cumsum(eq.astype(jnp.int32))
            sel = jnp.logical_or(gt, jnp.logical_and(eq, eq_rank <= r_tie))
            pos = off_o + plsc.cumsum(sel.astype(jnp.int32)) - 1
            plsc.store_scatter(outv_v, [pos], x, mask=sel)
            plsc.store_scatter(outi_v, [pos], cidx, mask=sel)
            return (off_o + plsc.all_reduce_population_count(sel),
                    eq_seen + plsc.all_reduce_population_count(eq))

        lax.fori_loop(0, nvc, fin, (zeros16, zeros16))

        def ce(ka, pa, kb, pb):
            c = ka >= kb
            return (jnp.where(c, ka, kb), jnp.where(c, pa, pb),
                    jnp.where(c, kb, ka), jnp.where(c, pb, pa))

        def rev(k, p):
            return lax.rev(k, (0,)), lax.rev(p, (0,))

        def perm_gather(k, p, pidx):
            sk_v[...] = k
            sp_v[...] = p
            return (plsc.load_gather(sk_v, [pidx]),
                    plsc.load_gather(sp_v, [pidx]))

        def clean(k, p):
            for j in (8, 4, 2, 1):
                pidx = lax.bitwise_xor(lanes, np.int32(j))
                kp, pp = perm_gather(k, p, pidx)
                is_lo = lax.bitwise_and(lanes, np.int32(j)) == 0
                keep = jnp.where(is_lo, k >= kp, kp >= k)
                k = jnp.where(keep, k, kp)
                p = jnp.where(keep, p, pp)
            return k, p

        def merge32(ka, pa, kb, pb):
            kb, pb = rev(kb, pb)
            ka, pa, kb, pb = ce(ka, pa, kb, pb)
            ka, pa = clean(ka, pa)
            kb, pb = clean(kb, pb)
            return ka, pa, kb, pb

        ks, ps = [], []
        for v in range(4):
            xv = outv_v[pl.ds(v * 16, 16)]
            kv, pv = plsc.sort_key_val(_key(xv), v * 16 + lanes,
                                       descending=True)
            ks.append(kv)
            ps.append(pv)

        ks[0], ps[0], ks[1], ps[1] = merge32(ks[0], ps[0], ks[1], ps[1])
        ks[2], ps[2], ks[3], ps[3] = merge32(ks[2], ps[2], ks[3], ps[3])

        rk3, rp3 = rev(ks[3], ps[3])
        rk2, rp2 = rev(ks[2], ps[2])
        k0, p0, rk3, rp3 = ce(ks[0], ps[0], rk3, rp3)
        k1, p1, rk2, rp2 = ce(ks[1], ps[1], rk2, rp2)
        k0, p0, k1, p1 = ce(k0, p0, k1, p1)
        rk3, rp3, rk2, rp2 = ce(rk3, rp3, rk2, rp2)
        k0, p0 = clean(k0, p0)
        k1, p1 = clean(k1, p1)
        k2, p2 = clean(rk3, rp3)
        k3, p3 = clean(rk2, rp2)

        kall = (k0, k1, k2, k3)
        for v in range(4):
            ks_v[pl.ds(v * 16, 16)] = kall[v]
        ps = [p0, p1, p2, p3]
        for q in (0, 1, 0, 1):
            for v in range(4):
                ps_v[pl.ds(v * 16, 16)] = ps[v]
            new_ps = []
            for v in range(4):
                e = v * 16 + lanes
                if q == 0:
                    partner = lax.bitwise_xor(e, 1)
                else:
                    partner = jnp.clip(lax.bitwise_xor(e + 1, 1) - 1, 0, 63)
                kp = plsc.load_gather(ks_v, [partner])
                pp = plsc.load_gather(ps_v, [partner])
                k, p = kall[v], ps[v]
                take = jnp.logical_and(
                    k == kp,
                    jnp.where(partner > e, pp < p, pp > p))
                new_ps.append(jnp.where(take, pp, p))
            ps = new_ps

        for v, pv in enumerate(ps):
            sortv_v[pl.ds(v * 16, 16)] = plsc.load_gather(outv_v, [pv])
            sorti_v[pl.ds(v * 16, 16)] = plsc.load_gather(outi_v, [pv])

        pltpu.sync_copy(sortv_v, vals_hbm.at[row])
        pltpu.sync_copy(sorti_v, idx_hbm.at[row])


@jax.jit
def _sc_topk(inputs):
    mesh = plsc.VectorSubcoreMesh(core_axis_name="c", subcore_axis_name="s")
    cp = pltpu.CompilerParams()
    if "needs_layout_passes" in pltpu.CompilerParams.__dataclass_fields__:
        cp = dataclasses.replace(cp, needs_layout_passes=False)
    f = pl.kernel(
        _topk_body,
        compiler_params=cp,
        out_type=[
            jax.ShapeDtypeStruct((N_ROWS, K), jnp.float32),
            jax.ShapeDtypeStruct((N_ROWS, K), jnp.int32),
        ],
        mesh=mesh,
        scratch_types=[
            pltpu.VMEM((N_COLS,), jnp.float32),
            pltpu.VMEM((N_COLS,), jnp.int32),
            pltpu.VMEM((256 * 16,), jnp.int32),
            pltpu.VMEM((K,), jnp.float32),
            pltpu.VMEM((K,), jnp.int32),
            pltpu.VMEM((16,), jnp.int32),
            pltpu.VMEM((16,), jnp.int32),
            pltpu.VMEM((K,), jnp.float32),
            pltpu.VMEM((K,), jnp.int32),
            pltpu.VMEM((K,), jnp.int32),
            pltpu.VMEM((K,), jnp.int32),
        ],
    )
    return f(inputs)


def kernel(inputs):
    vals, idxs = _sc_topk(inputs)
    return (vals, idxs)


# speedup vs baseline: 8.1236x; 2.5617x over previous
import dataclasses
import functools

import jax
import jax.numpy as jnp
import numpy as np
from jax import lax
from jax.experimental import pallas as pl
from jax.experimental.pallas import tpu as pltpu
from jax.experimental.pallas import tpu_sc as plsc

N_ROWS = 128
N_COLS = 32768
K = 64
NC = 2
NS = 16
NW = NC * NS
RPW = N_ROWS // NW
NV = N_COLS // 16
SIGN = np.int32(-2147483648)
MAGN = np.int32(0x7FFFFFFF)


def _key(x):
    u = plsc.bitcast(x, jnp.int32)
    s = lax.shift_right_arithmetic(u, 31)
    return lax.bitwise_xor(u, lax.bitwise_and(s, MAGN))


def _topk_body(x_hbm, vals_hbm, idx_hbm,
               raw_v, cand_v, hist_v, outv_v, outi_v,
               sk_v, sp_v, sortv_v, sorti_v, ks_v, ps_v):
    cid = lax.axis_index("c")
    sid = lax.axis_index("s")
    wid = sid * NC + cid
    lanes = lax.iota(jnp.int32, 16)
    zeros16 = lanes - lanes
    ones = zeros16 + 1

    def zero_hist():
        @pl.loop(0, 256)
        def _z(i):
            hist_v[pl.ds(i * 16, 16)] = zeros16

    def sweep(r):
        def cond(c):
            _, acc, _ = c
            return acc < r

        def body(c):
            b, acc, _ = c
            b = b - 1
            t = jnp.sum(hist_v[pl.ds(b * 16, 16)])
            return b, acc + t, t

        b, acc, t = lax.while_loop(
            cond, body, (np.int32(256), np.int32(0), np.int32(0)))
        return b, acc - t, t

    @pl.loop(0, RPW)
    def _row(t):
        row = wid * RPW + t
        pltpu.sync_copy(x_hbm.at[row], raw_v)

        zero_hist()

        @plsc.parallel_loop(0, NV, unroll=8)
        def _s1(i):
            x = raw_v[pl.ds(i * 16, 16)]
            kb = lax.bitwise_xor(_key(x), SIGN)
            d = lax.shift_right_logical(kb, 24)
            plsc.addupdate_scatter(hist_v, [d * 16 + lanes], ones)

        b1, g1, t1 = sweep(np.int32(K))

        @plsc.parallel_loop(0, NV, unroll=4, carry=zeros16)
        def off(i, off_c):
            x = raw_v[pl.ds(i * 16, 16)]
            kb = lax.bitwise_xor(_key(x), SIGN)
            d = lax.shift_right_logical(kb, 24)
            m = d >= b1
            pos = off_c + plsc.cumsum(m.astype(jnp.int32)) - 1
            plsc.store_scatter(cand_v, [pos], i * 16 + lanes, mask=m)
            return off_c + plsc.all_reduce_population_count(m)
        n = jnp.max(off)
        nvc = lax.shift_right_logical(n + 15, 4)

        def refine(B, r, cnt, shift):
            zero_hist()

            @plsc.parallel_loop(0, nvc, unroll=2)
            def _rb(i):
                base = i * 16
                valid = (base + lanes) < n
                cidx = cand_v[pl.ds(base, 16)]
                x = plsc.load_gather(raw_v, [cidx], mask=valid)
                kb = lax.bitwise_xor(_key(x), SIGN)
                pref = lax.shift_right_logical(kb, shift + 8)
                tie = jnp.logical_and(pref == B, valid)
                d = lax.bitwise_and(
                    lax.shift_right_logical(kb, shift), np.int32(0xFF))
                plsc.addupdate_scatter(hist_v, [d * 16 + lanes], ones, mask=tie)
            b, g, tb = sweep(r)
            return B * 256 + b, r - g, tb

        B, r, cnt = b1, np.int32(K) - g1, t1
        for shift in (16, 8, 0):
            def _skip(B, r, cnt):
                return B * 256, r, cnt

            def _do(B, r, cnt, _s=shift):
                return refine(B, r, cnt, _s)

            B, r, cnt = lax.cond(r == cnt, _skip, _do, B, r, cnt)

        take_all = r == cnt
        ts = lax.bitwise_xor(B, SIGN)
        tcut = jnp.where(take_all, ts - 1, ts)
        r_tie = jnp.where(take_all, 0, r)

        @plsc.parallel_loop(0, nvc, unroll=2, carry=(zeros16, zeros16))
        def _fin(i, carry):
            off_o, eq_seen = carry
            base = i * 16
            valid = (base + lanes) < n
            cidx = cand_v[pl.ds(base, 16)]
            x = plsc.load_gather(raw_v, [cidx], mask=valid)
            ms = _key(x)
            gt = jnp.logical_and(ms > tcut, valid)
            eq = jnp.logical_and(jnp.logical_and(ms == ts, valid),
                                 jnp.logical_not(gt))
            eq_rank = eq_seen + plsc.cumsum(eq.astype(jnp.int32))
            sel = jnp.logical_or(gt, jnp.logical_and(eq, eq_rank <= r_tie))
            pos = off_o + plsc.cumsum(sel.astype(jnp.int32)) - 1
            plsc.store_scatter(outv_v, [pos], x, mask=sel)
            plsc.store_scatter(outi_v, [pos], cidx, mask=sel)
            return (off_o + plsc.all_reduce_population_count(sel),
                    eq_seen + plsc.all_reduce_population_count(eq))

        def ce(ka, pa, kb, pb):
            c = ka >= kb
            return (jnp.where(c, ka, kb), jnp.where(c, pa, pb),
                    jnp.where(c, kb, ka), jnp.where(c, pb, pa))

        def rev(k, p):
            return lax.rev(k, (0,)), lax.rev(p, (0,))

        def perm_gather(k, p, pidx):
            sk_v[...] = k
            sp_v[...] = p
            return (plsc.load_gather(sk_v, [pidx]),
                    plsc.load_gather(sp_v, [pidx]))

        def clean(k, p):
            for j in (8, 4, 2, 1):
                pidx = lax.bitwise_xor(lanes, np.int32(j))
                kp, pp = perm_gather(k, p, pidx)
                is_lo = lax.bitwise_and(lanes, np.int32(j)) == 0
                keep = jnp.where(is_lo, k >= kp, kp >= k)
                k = jnp.where(keep, k, kp)
                p = jnp.where(keep, p, pp)
            return k, p

        def merge32(ka, pa, kb, pb):
            kb, pb = rev(kb, pb)
            ka, pa, kb, pb = ce(ka, pa, kb, pb)
            ka, pa = clean(ka, pa)
            kb, pb = clean(kb, pb)
            return ka, pa, kb, pb

        ks, ps = [], []
        for v in range(4):
            xv = outv_v[pl.ds(v * 16, 16)]
            kv, pv = plsc.sort_key_val(_key(xv), v * 16 + lanes,
                                       descending=True)
            ks.append(kv)
            ps.append(pv)

        ks[0], ps[0], ks[1], ps[1] = merge32(ks[0], ps[0], ks[1], ps[1])
        ks[2], ps[2], ks[3], ps[3] = merge32(ks[2], ps[2], ks[3], ps[3])

        rk3, rp3 = rev(ks[3], ps[3])
        rk2, rp2 = rev(ks[2], ps[2])
        k0, p0, rk3, rp3 = ce(ks[0], ps[0], rk3, rp3)
        k1, p1, rk2, rp2 = ce(ks[1], ps[1], rk2, rp2)
        k0, p0, k1, p1 = ce(k0, p0, k1, p1)
        rk3, rp3, rk2, rp2 = ce(rk3, rp3, rk2, rp2)
        k0, p0 = clean(k0, p0)
        k1, p1 = clean(k1, p1)
        k2, p2 = clean(rk3, rp3)
        k3, p3 = clean(rk2, rp2)

        kall = (k0, k1, k2, k3)
        for v in range(4):
            ks_v[pl.ds(v * 16, 16)] = kall[v]
        ps = [p0, p1, p2, p3]
        for q in (0, 1, 0, 1):
            for v in range(4):
                ps_v[pl.ds(v * 16, 16)] = ps[v]
            new_ps = []
            for v in range(4):
                e = v * 16 + lanes
                if q == 0:
                    partner = lax.bitwise_xor(e, 1)
                else:
                    partner = jnp.clip(lax.bitwise_xor(e + 1, 1) - 1, 0, 63)
                kp = plsc.load_gather(ks_v, [partner])
                pp = plsc.load_gather(ps_v, [partner])
                k, p = kall[v], ps[v]
                take = jnp.logical_and(
                    k == kp,
                    jnp.where(partner > e, pp < p, pp > p))
                new_ps.append(jnp.where(take, pp, p))
            ps = new_ps

        for v, pv in enumerate(ps):
            sortv_v[pl.ds(v * 16, 16)] = plsc.load_gather(outv_v, [pv])
            sorti_v[pl.ds(v * 16, 16)] = plsc.load_gather(outi_v, [pv])

        pltpu.sync_copy(sortv_v, vals_hbm.at[row])
        pltpu.sync_copy(sorti_v, idx_hbm.at[row])


@jax.jit
def _sc_topk(inputs):
    mesh = plsc.VectorSubcoreMesh(core_axis_name="c", subcore_axis_name="s")
    cp = pltpu.CompilerParams()
    if "needs_layout_passes" in pltpu.CompilerParams.__dataclass_fields__:
        cp = dataclasses.replace(cp, needs_layout_passes=False)
    f = pl.kernel(
        _topk_body,
        compiler_params=cp,
        out_type=[
            jax.ShapeDtypeStruct((N_ROWS, K), jnp.float32),
            jax.ShapeDtypeStruct((N_ROWS, K), jnp.int32),
        ],
        mesh=mesh,
        scratch_types=[
            pltpu.VMEM((N_COLS,), jnp.float32),
            pltpu.VMEM((N_COLS,), jnp.int32),
            pltpu.VMEM((256 * 16,), jnp.int32),
            pltpu.VMEM((K,), jnp.float32),
            pltpu.VMEM((K,), jnp.int32),
            pltpu.VMEM((16,), jnp.int32),
            pltpu.VMEM((16,), jnp.int32),
            pltpu.VMEM((K,), jnp.float32),
            pltpu.VMEM((K,), jnp.int32),
            pltpu.VMEM((K,), jnp.int32),
            pltpu.VMEM((K,), jnp.int32),
        ],
    )
    return f(inputs)


def kernel(inputs):
    vals, idxs = _sc_topk(inputs)
    return (vals, idxs)
